# Initial kernel scaffold; baseline (speedup 1.0000x reference)
import jax
import jax.numpy as jnp
from jax.experimental import pallas as pl


def kernel(seq, embed, W1, b1, W2, b2, gamma, beta, Wg, bg, Wq, bq, Wo, bo):
    raise NotImplementedError("write your pallas kernel here")


# speedup vs baseline: 1.2168x; 1.2168x over previous
import functools

import jax
import jax.numpy as jnp
from jax import lax
from jax.experimental import pallas as pl
from jax.experimental.pallas import tpu as pltpu
from jax.experimental.pallas import tpu_sc as plsc

VOCAB = 32000
HID = 1024
T = 2048
BATCH = 4
NTOK = BATCH * T
K = 256
NCAND = T - 3
NC, NS, L = 2, 16, 16
NW = NC * NS

_PREC = lax.Precision.DEFAULT


def _r16(x):
    return x.astype(jnp.bfloat16).astype(jnp.float32)


def _embed_gather(seq3, embed):
    n_ch, ch = seq3.shape[1], seq3.shape[2]
    tok_per_w = n_ch * ch
    mesh = plsc.VectorSubcoreMesh(core_axis_name="c", subcore_axis_name="s")

    @functools.partial(
        pl.kernel,
        out_type=jax.ShapeDtypeStruct((NTOK, HID), jnp.float32),
        mesh=mesh,
        scratch_types=[
            pltpu.VMEM((n_ch, ch), jnp.int32),
            pltpu.VMEM((ch, HID), jnp.float32),
            pltpu.VMEM((ch, HID), jnp.float32),
            pltpu.SemaphoreType.DMA,
            pltpu.SemaphoreType.DMA,
            pltpu.SemaphoreType.DMA,
            pltpu.SemaphoreType.DMA,
        ],
        compiler_params=pltpu.CompilerParams(needs_layout_passes=False),
    )
    def k(seq_hbm, embed_hbm, h_hbm, idx_v, rows0, rows1, g0, g1, s0, s1):
        wid = lax.axis_index("s") * NC + lax.axis_index("c")
        base = wid * tok_per_w
        pltpu.sync_copy(seq_hbm.at[wid], idx_v)
        rows = (rows0, rows1)
        gsem = (g0, g1)
        ssem = (s0, s1)
        cps = [None, None]
        outs = [None, None]
        cps[0] = pltpu.async_copy(embed_hbm.at[idx_v.at[0]], rows[0], gsem[0])
        for j in range(n_ch):
            p = j & 1
            if j + 1 < n_ch:
                q = (j + 1) & 1
                if outs[q] is not None:
                    outs[q].wait()
                cps[q] = pltpu.async_copy(
                    embed_hbm.at[idx_v.at[j + 1]], rows[q], gsem[q])
            cps[p].wait()
            outs[p] = pltpu.async_copy(
                rows[p], h_hbm.at[pl.ds(base + j * ch, ch)], ssem[p])
        for o in outs:
            if o is not None:
                o.wait()

    return k(seq3, embed)


def _encoder(h, W1, b1r, W2, b2r, gammar, betar, wgr):
    BT = 256
    grid = (NTOK // BT,)

    def body(h_ref, w1_ref, b1_ref, w2_ref, b2_ref, g_ref, be_ref, wg_ref,
             hid_ref, sc_ref):
        hb = h_ref[...]
        a1 = lax.dot_general(hb, w1_ref[...], (((1,), (0,)), ((), ())),
                             preferred_element_type=jnp.float32, precision=_PREC)
        a1 = jnp.maximum(a1 + b1_ref[...], 0.0)
        x = lax.dot_general(a1, w2_ref[...], (((1,), (0,)), ((), ())),
                            preferred_element_type=jnp.float32, precision=_PREC)
        x = x + b2_ref[...] + hb
        mu = jnp.mean(x, axis=-1, keepdims=True)
        xc = x - mu
        var = jnp.mean(xc * xc, axis=-1, keepdims=True)
        ln = xc * lax.rsqrt(var + 1e-5) * g_ref[...] + be_ref[...]
        hid_ref[...] = ln
        sc_ref[...] = jnp.sum(_r16(ln) * _r16(wg_ref[...]), axis=-1,
                              keepdims=True)

    hidden, scores = pl.pallas_call(
        body,
        grid=grid,
        in_specs=[
            pl.BlockSpec((BT, HID), lambda i: (i, 0)),
            pl.BlockSpec((HID, 2 * HID), lambda i: (0, 0)),
            pl.BlockSpec((1, 2 * HID), lambda i: (0, 0)),
            pl.BlockSpec((2 * HID, HID), lambda i: (0, 0)),
            pl.BlockSpec((1, HID), lambda i: (0, 0)),
            pl.BlockSpec((1, HID), lambda i: (0, 0)),
            pl.BlockSpec((1, HID), lambda i: (0, 0)),
            pl.BlockSpec((1, HID), lambda i: (0, 0)),
        ],
        out_specs=[
            pl.BlockSpec((BT, HID), lambda i: (i, 0)),
            pl.BlockSpec((BT, 1), lambda i: (i, 0)),
        ],
        out_shape=[
            jax.ShapeDtypeStruct((NTOK, HID), jnp.float32),
            jax.ShapeDtypeStruct((NTOK, 1), jnp.float32),
        ],
    )(h, W1, b1r, W2, b2r, gammar, betar, wgr)
    return hidden, scores


def _topk_gather(scores, hidden):
    mesh = plsc.VectorSubcoreMesh(core_axis_name="c", subcore_axis_name="s")
    NV = T // L
    GROWS = K // 8

    @functools.partial(
        pl.kernel,
        out_type=jax.ShapeDtypeStruct((BATCH, K, HID), jnp.float32),
        mesh=mesh,
        scratch_types=[
            pltpu.VMEM((T,), jnp.float32),
            pltpu.VMEM((T,), jnp.uint32),
            pltpu.VMEM((T,), jnp.int32),
            pltpu.VMEM((16 * L,), jnp.int32),
            pltpu.VMEM((K,), jnp.int32),
            pltpu.VMEM((GROWS,), jnp.int32),
            pltpu.VMEM((GROWS, HID), jnp.float32),
            pltpu.VMEM_SHARED((BATCH, K), jnp.int32),
            pltpu.SemaphoreType.DMA,
        ],
        compiler_params=pltpu.CompilerParams(needs_layout_passes=False),
    )
    def k(sc_hbm, hid_hbm, mem_hbm, srow, keys, act, hist, selidx, gidx,
          mrows, shared_idx, sem):
        cid = lax.axis_index("c")
        sid = lax.axis_index("s")
        lane = lax.iota(jnp.int32, L)
        zero16 = jnp.zeros((L,), jnp.int32)
        ones16 = jnp.ones((L,), jnp.int32)

        @pl.when(sid < BATCH)
        def _phase_a():
            b = sid
            pltpu.sync_copy(sc_hbm.at[b], srow)

            def mkkeys(i, _):
                sv = srow[pl.ds(i * L, L)]
                ki = plsc.bitcast(sv, jnp.int32)
                sgn = lax.shift_right_arithmetic(ki, 31)
                key = plsc.bitcast(ki, jnp.uint32) ^ (
                    plsc.bitcast(sgn, jnp.uint32) | jnp.uint32(0x80000000))
                valid = (i * L + lane) < NCAND
                keys[pl.ds(i * L, L)] = jnp.where(valid, key, jnp.uint32(0))
                return 0
            lax.fori_loop(0, NV, mkkeys, 0)

            tkey = jnp.uint32(0)
            target = jnp.int32(K)
            n_act = jnp.int32(T)
            for p in range(8):
                sh = 28 - 4 * p
                src = keys if p == 0 else act

                def zh(i, _):
                    hist[pl.ds(i * L, L)] = zero16
                    return 0
                lax.fori_loop(0, 16, zh, 0)

                def acc(i, _, n_act=n_act, src=src, sh=sh, p=p):
                    kv = src[pl.ds(i * L, L)]
                    if p > 0:
                        kv = plsc.bitcast(kv, jnp.uint32)
                    in_rng = (i * L + lane) < n_act
                    dig = lax.convert_element_type(
                        lax.shift_right_logical(kv, jnp.uint32(sh))
                        & jnp.uint32(0xF), jnp.int32)
                    plsc.addupdate_scatter(hist, [dig * L + lane], ones16,
                                           mask=in_rng)
                    return 0
                nvec = NV if p == 0 else (n_act + L - 1) // L
                lax.fori_loop(0, nvec, acc, 0)

                def scan(i, c, target=target):
                    cnt_above, bsel, m = c
                    bn = 15 - i
                    cv = jnp.sum(hist[pl.ds(bn * L, L)])
                    hit = (cnt_above < target) & (cnt_above + cv >= target)
                    bsel = jnp.where(hit, bn, bsel)
                    m = jnp.where(hit, target - cnt_above, m)
                    return (cnt_above + cv, bsel, m)
                _, bsel, m = lax.fori_loop(0, 16, scan,
                                           (jnp.int32(0), jnp.int32(0),
                                            jnp.int32(0)))
                bu = lax.convert_element_type(bsel, jnp.uint32)

                if p < 7:
                    def comp(i, cnt, n_act=n_act, src=src, sh=sh, bu=bu, p=p):
                        kv = src[pl.ds(i * L, L)]
                        if p > 0:
                            kv = plsc.bitcast(kv, jnp.uint32)
                        in_rng = (i * L + lane) < n_act
                        dig = lax.shift_right_logical(kv, jnp.uint32(sh)) \
                            & jnp.uint32(0xF)
                        sel = in_rng & (dig == bu)
                        si = jnp.where(sel, 1, 0)
                        dst = cnt + plsc.cumsum(si) - si
                        plsc.store_scatter(act, [dst],
                                           plsc.bitcast(kv, jnp.int32),
                                           mask=sel)
                        return cnt + jnp.sum(si)
                    n_act = lax.fori_loop(0, nvec, comp, jnp.int32(0))
                tkey = tkey | lax.shift_left(bu, jnp.uint32(sh))
                target = m
            m = target

            def sel(i, c):
                n_sel, n_eq = c
                kv = keys[pl.ds(i * L, L)]
                gt = kv > tkey
                eq = kv == tkey
                eqi = jnp.where(eq, 1, 0)
                pre = plsc.cumsum(eqi) - eqi
                take = gt | (eq & ((n_eq + pre) < m))
                ti = jnp.where(take, 1, 0)
                dst = n_sel + plsc.cumsum(ti) - ti
                gidx16 = b * T + i * L + lane
                plsc.store_scatter(selidx, [dst], gidx16, mask=take)
                return (n_sel + jnp.sum(ti), n_eq + jnp.sum(eqi))
            lax.fori_loop(0, NV, sel, (0, 0))
            pltpu.sync_copy(selidx, shared_idx.at[b])

        plsc.subcore_barrier()

        g = sid * NC + cid
        b2 = g // 8
        part = (g % 8) * GROWS
        pltpu.sync_copy(shared_idx.at[b2, pl.ds(part, GROWS)], gidx)
        pltpu.async_copy(hid_hbm.at[gidx], mrows, sem).wait()
        pltpu.sync_copy(mrows, mem_hbm.at[b2, pl.ds(part, GROWS)])

    return k(scores, hidden)


def _attn_logits(memory, query, Wq, bqr, Wo, bor):
    NVB = 3200
    grid = (VOCAB // NVB,)

    def body(mem_ref, q_ref, wq_ref, bq_ref, wo_ref, bo_ref, out_ref):
        q = lax.dot_general(q_ref[...], wq_ref[...], (((1,), (0,)), ((), ())),
                            preferred_element_type=jnp.float32,
                            precision=_PREC) + bq_ref[...]
        ctxs = []
        for b in range(BATCH):
            mb = mem_ref[b]
            qb = q[b:b + 1, :]
            s = jnp.sum(_r16(mb) * _r16(qb), axis=-1, keepdims=True)
            s = s - jnp.max(s, axis=0, keepdims=True)
            e = jnp.exp(s)
            attn = e / jnp.sum(e, axis=0, keepdims=True)
            ctxs.append(jnp.sum(attn * mb, axis=0, keepdims=True))
        ctx = jnp.concatenate(ctxs, axis=0)
        out_ref[...] = lax.dot_general(
            ctx, wo_ref[...], (((1,), (0,)), ((), ())),
            preferred_element_type=jnp.float32, precision=_PREC) + bo_ref[...]

    return pl.pallas_call(
        body,
        grid=grid,
        in_specs=[
            pl.BlockSpec((BATCH, K, HID), lambda i: (0, 0, 0)),
            pl.BlockSpec((BATCH, HID), lambda i: (0, 0)),
            pl.BlockSpec((HID, HID), lambda i: (0, 0)),
            pl.BlockSpec((1, HID), lambda i: (0, 0)),
            pl.BlockSpec((HID, NVB), lambda i: (0, i)),
            pl.BlockSpec((1, NVB), lambda i: (0, i)),
        ],
        out_specs=pl.BlockSpec((BATCH, NVB), lambda i: (0, i)),
        out_shape=jax.ShapeDtypeStruct((BATCH, VOCAB), jnp.float32),
    )(memory, query, Wq, bqr, Wo, bor)


def kernel(seq, embed, W1, b1, W2, b2, gamma, beta, Wg, bg, Wq, bq, Wo, bo):
    seq3 = seq.astype(jnp.int32).reshape(NW, 8, NTOK // NW // 8)
    h = _embed_gather(seq3, embed)
    hidden, scores = _encoder(
        h, W1, b1.reshape(1, -1), W2, b2.reshape(1, -1),
        gamma.reshape(1, -1), beta.reshape(1, -1), Wg.reshape(1, -1))
    memory = _topk_gather(scores.reshape(BATCH, T), hidden)
    query = hidden.reshape(BATCH, T, HID)[:, -2, :]
    return _attn_logits(memory, query, Wq, bq.reshape(1, -1),
                        Wo, bo.reshape(1, -1))


# speedup vs baseline: 1.2206x; 1.0031x over previous
import functools

import jax
import jax.numpy as jnp
from jax import lax
from jax.experimental import pallas as pl
from jax.experimental.pallas import tpu as pltpu
from jax.experimental.pallas import tpu_sc as plsc

VOCAB = 32000
HID = 1024
T = 2048
BATCH = 4
NTOK = BATCH * T
K = 256
NCAND = T - 3
NC, NS, L = 2, 16, 16
NW = NC * NS

_PREC = lax.Precision.DEFAULT


def _r16(x):
    return x.astype(jnp.bfloat16).astype(jnp.float32)


def _embed_gather(seq3, embed):
    n_ch, ch = seq3.shape[1], seq3.shape[2]
    tok_per_w = n_ch * ch
    mesh = plsc.VectorSubcoreMesh(core_axis_name="c", subcore_axis_name="s")

    @functools.partial(
        pl.kernel,
        out_type=jax.ShapeDtypeStruct((NTOK, HID), jnp.float32),
        mesh=mesh,
        scratch_types=[
            pltpu.VMEM((n_ch, ch), jnp.int32),
            pltpu.VMEM((ch, HID), jnp.float32),
            pltpu.VMEM((ch, HID), jnp.float32),
            pltpu.SemaphoreType.DMA,
            pltpu.SemaphoreType.DMA,
            pltpu.SemaphoreType.DMA,
            pltpu.SemaphoreType.DMA,
        ],
        compiler_params=pltpu.CompilerParams(needs_layout_passes=False),
    )
    def k(seq_hbm, embed_hbm, h_hbm, idx_v, rows0, rows1, g0, g1, s0, s1):
        wid = lax.axis_index("s") * NC + lax.axis_index("c")
        base = wid * tok_per_w
        pltpu.sync_copy(seq_hbm.at[wid], idx_v)
        rows = (rows0, rows1)
        gsem = (g0, g1)
        ssem = (s0, s1)
        cps = [None, None]
        outs = [None, None]
        cps[0] = pltpu.async_copy(embed_hbm.at[idx_v.at[0]], rows[0], gsem[0])
        for j in range(n_ch):
            p = j & 1
            if j + 1 < n_ch:
                q = (j + 1) & 1
                if outs[q] is not None:
                    outs[q].wait()
                cps[q] = pltpu.async_copy(
                    embed_hbm.at[idx_v.at[j + 1]], rows[q], gsem[q])
            cps[p].wait()
            outs[p] = pltpu.async_copy(
                rows[p], h_hbm.at[pl.ds(base + j * ch, ch)], ssem[p])
        for o in outs:
            if o is not None:
                o.wait()

    return k(seq3, embed)


def _encoder(h, W1, b1r, W2, b2r, gammar, betar, wgr):
    BT = 256
    grid = (NTOK // BT,)

    def body(h_ref, w1_ref, b1_ref, w2_ref, b2_ref, g_ref, be_ref, wg_ref,
             hid_ref, sc_ref):
        hb = h_ref[...]
        a1 = lax.dot_general(hb, w1_ref[...], (((1,), (0,)), ((), ())),
                             preferred_element_type=jnp.float32,
                             precision=_PREC)
        a1 = jnp.maximum(a1 + b1_ref[...], 0.0)
        x = lax.dot_general(a1, w2_ref[...], (((1,), (0,)), ((), ())),
                            preferred_element_type=jnp.float32,
                            precision=_PREC)
        x = x + b2_ref[...] + hb
        mu = jnp.mean(x, axis=-1, keepdims=True)
        xc = x - mu
        var = jnp.mean(xc * xc, axis=-1, keepdims=True)
        ln = xc * lax.rsqrt(var + 1e-5) * g_ref[...] + be_ref[...]
        hid_ref[...] = ln
        sc_ref[...] = jnp.sum(_r16(ln) * _r16(wg_ref[...]), axis=-1,
                              keepdims=True)

    hidden, scores = pl.pallas_call(
        body,
        grid=grid,
        in_specs=[
            pl.BlockSpec((BT, HID), lambda i: (i, 0)),
            pl.BlockSpec((HID, 2 * HID), lambda i: (0, 0)),
            pl.BlockSpec((1, 2 * HID), lambda i: (0, 0)),
            pl.BlockSpec((2 * HID, HID), lambda i: (0, 0)),
            pl.BlockSpec((1, HID), lambda i: (0, 0)),
            pl.BlockSpec((1, HID), lambda i: (0, 0)),
            pl.BlockSpec((1, HID), lambda i: (0, 0)),
            pl.BlockSpec((1, HID), lambda i: (0, 0)),
        ],
        out_specs=[
            pl.BlockSpec((BT, HID), lambda i: (i, 0)),
            pl.BlockSpec((BT, 1), lambda i: (i, 0)),
        ],
        out_shape=[
            jax.ShapeDtypeStruct((NTOK, HID), jnp.float32),
            jax.ShapeDtypeStruct((NTOK, 1), jnp.float32),
        ],
    )(h, W1, b1r, W2, b2r, gammar, betar, wgr)
    return hidden, scores


def _topk_gather(scores, hidden):
    mesh = plsc.VectorSubcoreMesh(core_axis_name="c", subcore_axis_name="s")
    NV = T // L
    GROWS = K // 8

    @functools.partial(
        pl.kernel,
        out_type=jax.ShapeDtypeStruct((BATCH, K, HID), jnp.float32),
        mesh=mesh,
        scratch_types=[
            pltpu.VMEM((T,), jnp.float32),
            pltpu.VMEM((T,), jnp.uint32),
            pltpu.VMEM((T,), jnp.int32),
            pltpu.VMEM((16 * L,), jnp.int32),
            pltpu.VMEM((K,), jnp.int32),
            pltpu.VMEM((GROWS,), jnp.int32),
            pltpu.VMEM((GROWS, HID), jnp.float32),
            pltpu.VMEM_SHARED((BATCH, K), jnp.int32),
            pltpu.SemaphoreType.DMA,
        ],
        compiler_params=pltpu.CompilerParams(needs_layout_passes=False),
    )
    def k(sc_hbm, hid_hbm, mem_hbm, srow, keys, act, hist, selidx, gidx,
          mrows, shared_idx, sem):
        cid = lax.axis_index("c")
        sid = lax.axis_index("s")
        lane = lax.iota(jnp.int32, L)
        zero16 = jnp.zeros((L,), jnp.int32)
        ones16 = jnp.ones((L,), jnp.int32)

        @pl.when(sid < BATCH)
        def _phase_a():
            b = sid
            pltpu.sync_copy(sc_hbm.at[b], srow)

            def mkkeys(i, _):
                sv = srow[pl.ds(i * L, L)]
                ki = plsc.bitcast(sv, jnp.int32)
                sgn = lax.shift_right_arithmetic(ki, 31)
                key = plsc.bitcast(ki, jnp.uint32) ^ (
                    plsc.bitcast(sgn, jnp.uint32) | jnp.uint32(0x80000000))
                valid = (i * L + lane) < NCAND
                keys[pl.ds(i * L, L)] = jnp.where(valid, key, jnp.uint32(0))
                return 0
            lax.fori_loop(0, NV, mkkeys, 0)

            tkey = jnp.uint32(0)
            target = jnp.int32(K)
            n_act = jnp.int32(T)
            for p in range(8):
                sh = 28 - 4 * p
                src = keys if p == 0 else act

                def zh(i, _):
                    hist[pl.ds(i * L, L)] = zero16
                    return 0
                lax.fori_loop(0, 16, zh, 0)

                def acc(i, _, n_act=n_act, src=src, sh=sh, p=p):
                    kv = src[pl.ds(i * L, L)]
                    if p > 0:
                        kv = plsc.bitcast(kv, jnp.uint32)
                    in_rng = (i * L + lane) < n_act
                    dig = lax.convert_element_type(
                        lax.shift_right_logical(kv, jnp.uint32(sh))
                        & jnp.uint32(0xF), jnp.int32)
                    plsc.addupdate_scatter(hist, [dig * L + lane], ones16,
                                           mask=in_rng)
                    return 0
                nvec = NV if p == 0 else (n_act + L - 1) // L
                lax.fori_loop(0, nvec, acc, 0)

                def scan(i, c, target=target):
                    cnt_above, bsel, m = c
                    bn = 15 - i
                    cv = jnp.sum(hist[pl.ds(bn * L, L)])
                    hit = (cnt_above < target) & (cnt_above + cv >= target)
                    bsel = jnp.where(hit, bn, bsel)
                    m = jnp.where(hit, target - cnt_above, m)
                    return (cnt_above + cv, bsel, m)
                _, bsel, m = lax.fori_loop(0, 16, scan,
                                           (jnp.int32(0), jnp.int32(0),
                                            jnp.int32(0)))
                bu = lax.convert_element_type(bsel, jnp.uint32)

                if p < 7:
                    def comp(i, cnt, n_act=n_act, src=src, sh=sh, bu=bu, p=p):
                        kv = src[pl.ds(i * L, L)]
                        if p > 0:
                            kv = plsc.bitcast(kv, jnp.uint32)
                        in_rng = (i * L + lane) < n_act
                        dig = lax.shift_right_logical(kv, jnp.uint32(sh)) \
                            & jnp.uint32(0xF)
                        sel = in_rng & (dig == bu)
                        si = jnp.where(sel, 1, 0)
                        dst = cnt + plsc.cumsum(si) - si
                        plsc.store_scatter(act, [dst],
                                           plsc.bitcast(kv, jnp.int32),
                                           mask=sel)
                        return cnt + jnp.sum(si)
                    n_act = lax.fori_loop(0, nvec, comp, jnp.int32(0))
                tkey = tkey | lax.shift_left(bu, jnp.uint32(sh))
                target = m
            m = target

            def sel(i, c):
                n_sel, n_eq = c
                kv = keys[pl.ds(i * L, L)]
                gt = kv > tkey
                eq = kv == tkey
                eqi = jnp.where(eq, 1, 0)
                pre = plsc.cumsum(eqi) - eqi
                take = gt | (eq & ((n_eq + pre) < m))
                ti = jnp.where(take, 1, 0)
                dst = n_sel + plsc.cumsum(ti) - ti
                gidx16 = b * T + i * L + lane
                plsc.store_scatter(selidx, [dst], gidx16, mask=take)
                return (n_sel + jnp.sum(ti), n_eq + jnp.sum(eqi))
            lax.fori_loop(0, NV, sel, (0, 0))
            pltpu.sync_copy(selidx, shared_idx.at[b])

        plsc.subcore_barrier()

        g = sid * NC + cid
        b2 = g // 8
        part = (g % 8) * GROWS
        pltpu.sync_copy(shared_idx.at[b2, pl.ds(part, GROWS)], gidx)
        pltpu.async_copy(hid_hbm.at[gidx], mrows, sem).wait()
        pltpu.sync_copy(mrows, mem_hbm.at[b2, pl.ds(part, GROWS)])

    return k(scores, hidden)


def _attn_logits(memory, query, Wq, bqr, Wo, bor):
    NVB = 3200
    grid = (VOCAB // NVB,)

    def body(mem_ref, q_ref, wq_ref, bq_ref, wo_ref, bo_ref, out_ref, ctx_ref):
        @pl.when(pl.program_id(0) == 0)
        def _():
            q = lax.dot_general(
                q_ref[...], wq_ref[...], (((1,), (0,)), ((), ())),
                preferred_element_type=jnp.float32,
                precision=_PREC) + bq_ref[...]
            ctxs = []
            for b in range(BATCH):
                mb = mem_ref[b]
                qb = q[b:b + 1, :]
                s = jnp.sum(_r16(mb) * _r16(qb), axis=-1,
                            keepdims=True)
                s = s - jnp.max(s, axis=0, keepdims=True)
                e = jnp.exp(s)
                attn = e / jnp.sum(e, axis=0, keepdims=True)
                ctxs.append(jnp.sum(attn * mb, axis=0, keepdims=True))
            ctx_ref[...] = jnp.concatenate(ctxs, axis=0)
        out_ref[...] = lax.dot_general(
            ctx_ref[...], wo_ref[...], (((1,), (0,)), ((), ())),
            preferred_element_type=jnp.float32,
            precision=_PREC) + bo_ref[...]

    return pl.pallas_call(
        body,
        grid=grid,
        in_specs=[
            pl.BlockSpec((BATCH, K, HID), lambda i: (0, 0, 0)),
            pl.BlockSpec((BATCH, HID), lambda i: (0, 0)),
            pl.BlockSpec((HID, HID), lambda i: (0, 0)),
            pl.BlockSpec((1, HID), lambda i: (0, 0)),
            pl.BlockSpec((HID, NVB), lambda i: (0, i)),
            pl.BlockSpec((1, NVB), lambda i: (0, i)),
        ],
        out_specs=pl.BlockSpec((BATCH, NVB), lambda i: (0, i)),
        out_shape=jax.ShapeDtypeStruct((BATCH, VOCAB), jnp.float32),
        scratch_shapes=[pltpu.VMEM((BATCH, HID), jnp.float32)],
    )(memory, query, Wq, bqr, Wo, bor)


def kernel(seq, embed, W1, b1, W2, b2, gamma, beta, Wg, bg, Wq, bq, Wo, bo):
    seq3 = seq.astype(jnp.int32).reshape(NW, 8, NTOK // NW // 8)
    h = _embed_gather(seq3, embed)
    hidden, scores = _encoder(
        h, W1, b1.reshape(1, -1), W2, b2.reshape(1, -1),
        gamma.reshape(1, -1), beta.reshape(1, -1), Wg.reshape(1, -1))
    memory = _topk_gather(scores.reshape(BATCH, T), hidden)
    query = hidden.reshape(BATCH, T, HID)[:, -2, :]
    return _attn_logits(memory, query, Wq, bq.reshape(1, -1),
                        Wo, bo.reshape(1, -1))


# speedup vs baseline: 1.2738x; 1.0436x over previous
import functools

import jax
import jax.numpy as jnp
from jax import lax
from jax.experimental import pallas as pl
from jax.experimental.pallas import tpu as pltpu
from jax.experimental.pallas import tpu_sc as plsc

VOCAB = 32000
HID = 1024
T = 2048
BATCH = 4
NTOK = BATCH * T
K = 256
NCAND = T - 3
NC, NS, L = 2, 16, 16
NW = NC * NS

_PREC = lax.Precision.DEFAULT


def _r16(x):
    return x.astype(jnp.bfloat16).astype(jnp.float32)


def _embed_gather(seq3, embed):
    n_ch, ch = seq3.shape[1], seq3.shape[2]
    tok_per_w = n_ch * ch
    mesh = plsc.VectorSubcoreMesh(core_axis_name="c", subcore_axis_name="s")

    @functools.partial(
        pl.kernel,
        out_type=jax.ShapeDtypeStruct((NTOK, HID), jnp.float32),
        mesh=mesh,
        scratch_types=[
            pltpu.VMEM((n_ch, ch), jnp.int32),
            pltpu.VMEM((ch, HID), jnp.float32),
            pltpu.VMEM((ch, HID), jnp.float32),
            pltpu.SemaphoreType.DMA,
            pltpu.SemaphoreType.DMA,
            pltpu.SemaphoreType.DMA,
            pltpu.SemaphoreType.DMA,
        ],
        compiler_params=pltpu.CompilerParams(needs_layout_passes=False),
    )
    def k(seq_hbm, embed_hbm, h_hbm, idx_v, rows0, rows1, g0, g1, s0, s1):
        wid = lax.axis_index("s") * NC + lax.axis_index("c")
        base = wid * tok_per_w
        pltpu.sync_copy(seq_hbm.at[wid], idx_v)
        rows = (rows0, rows1)
        gsem = (g0, g1)
        ssem = (s0, s1)
        cps = [None, None]
        outs = [None, None]
        cps[0] = pltpu.async_copy(embed_hbm.at[idx_v.at[0]], rows[0], gsem[0])
        for j in range(n_ch):
            p = j & 1
            if j + 1 < n_ch:
                q = (j + 1) & 1
                if outs[q] is not None:
                    outs[q].wait()
                cps[q] = pltpu.async_copy(
                    embed_hbm.at[idx_v.at[j + 1]], rows[q], gsem[q])
            cps[p].wait()
            outs[p] = pltpu.async_copy(
                rows[p], h_hbm.at[pl.ds(base + j * ch, ch)], ssem[p])
        for o in outs:
            if o is not None:
                o.wait()

    return k(seq3, embed)


def _encoder(h, W1, b1r, W2, b2r, gammar, betar, wgr):
    BT = 1024
    grid = (NTOK // BT,)

    def body(h_ref, w1_ref, b1_ref, w2_ref, b2_ref, g_ref, be_ref, wg_ref,
             hid_ref, sc_ref):
        hb = h_ref[...]
        a1 = lax.dot_general(hb, w1_ref[...], (((1,), (0,)), ((), ())),
                             preferred_element_type=jnp.float32,
                             precision=_PREC)
        a1 = jnp.maximum(a1 + b1_ref[...], 0.0)
        x = lax.dot_general(a1, w2_ref[...], (((1,), (0,)), ((), ())),
                            preferred_element_type=jnp.float32,
                            precision=_PREC)
        x = x + b2_ref[...] + hb
        mu = jnp.mean(x, axis=-1, keepdims=True)
        xc = x - mu
        var = jnp.mean(xc * xc, axis=-1, keepdims=True)
        ln = xc * lax.rsqrt(var + 1e-5) * g_ref[...] + be_ref[...]
        hid_ref[...] = ln
        sc_ref[...] = jnp.sum(_r16(ln) * _r16(wg_ref[...]), axis=-1,
                              keepdims=True)

    hidden, scores = pl.pallas_call(
        body,
        grid=grid,
        in_specs=[
            pl.BlockSpec((BT, HID), lambda i: (i, 0)),
            pl.BlockSpec((HID, 2 * HID), lambda i: (0, 0)),
            pl.BlockSpec((1, 2 * HID), lambda i: (0, 0)),
            pl.BlockSpec((2 * HID, HID), lambda i: (0, 0)),
            pl.BlockSpec((1, HID), lambda i: (0, 0)),
            pl.BlockSpec((1, HID), lambda i: (0, 0)),
            pl.BlockSpec((1, HID), lambda i: (0, 0)),
            pl.BlockSpec((1, HID), lambda i: (0, 0)),
        ],
        out_specs=[
            pl.BlockSpec((BT, HID), lambda i: (i, 0)),
            pl.BlockSpec((BT, 1), lambda i: (i, 0)),
        ],
        out_shape=[
            jax.ShapeDtypeStruct((NTOK, HID), jnp.float32),
            jax.ShapeDtypeStruct((NTOK, 1), jnp.float32),
        ],
    )(h, W1, b1r, W2, b2r, gammar, betar, wgr)
    return hidden, scores


def _topk_gather(scores, hidden):
    mesh = plsc.VectorSubcoreMesh(core_axis_name="c", subcore_axis_name="s")
    NV = T // L
    GROWS = K // 8

    @functools.partial(
        pl.kernel,
        out_type=jax.ShapeDtypeStruct((BATCH, K, HID), jnp.float32),
        mesh=mesh,
        scratch_types=[
            pltpu.VMEM((T,), jnp.float32),
            pltpu.VMEM((T,), jnp.uint32),
            pltpu.VMEM((T,), jnp.int32),
            pltpu.VMEM((16 * L,), jnp.int32),
            pltpu.VMEM((K,), jnp.int32),
            pltpu.VMEM((GROWS,), jnp.int32),
            pltpu.VMEM((GROWS, HID), jnp.float32),
            pltpu.VMEM_SHARED((BATCH, K), jnp.int32),
            pltpu.SemaphoreType.DMA,
        ],
        compiler_params=pltpu.CompilerParams(needs_layout_passes=False),
    )
    def k(sc_hbm, hid_hbm, mem_hbm, srow, keys, act, hist, selidx, gidx,
          mrows, shared_idx, sem):
        cid = lax.axis_index("c")
        sid = lax.axis_index("s")
        lane = lax.iota(jnp.int32, L)
        zero16 = jnp.zeros((L,), jnp.int32)
        ones16 = jnp.ones((L,), jnp.int32)

        @pl.when(sid < BATCH)
        def _phase_a():
            b = sid
            pltpu.sync_copy(sc_hbm.at[b], srow)

            def mkkeys(i, _):
                sv = srow[pl.ds(i * L, L)]
                ki = plsc.bitcast(sv, jnp.int32)
                sgn = lax.shift_right_arithmetic(ki, 31)
                key = plsc.bitcast(ki, jnp.uint32) ^ (
                    plsc.bitcast(sgn, jnp.uint32) | jnp.uint32(0x80000000))
                valid = (i * L + lane) < NCAND
                keys[pl.ds(i * L, L)] = jnp.where(valid, key, jnp.uint32(0))
                return 0
            lax.fori_loop(0, NV, mkkeys, 0)

            tkey = jnp.uint32(0)
            target = jnp.int32(K)
            n_act = jnp.int32(T)
            for p in range(8):
                sh = 28 - 4 * p
                src = keys if p == 0 else act

                def zh(i, _):
                    hist[pl.ds(i * L, L)] = zero16
                    return 0
                lax.fori_loop(0, 16, zh, 0)

                def acc(i, _, n_act=n_act, src=src, sh=sh, p=p):
                    kv = src[pl.ds(i * L, L)]
                    if p > 0:
                        kv = plsc.bitcast(kv, jnp.uint32)
                    in_rng = (i * L + lane) < n_act
                    dig = lax.convert_element_type(
                        lax.shift_right_logical(kv, jnp.uint32(sh))
                        & jnp.uint32(0xF), jnp.int32)
                    plsc.addupdate_scatter(hist, [dig * L + lane], ones16,
                                           mask=in_rng)
                    return 0
                nvec = NV if p == 0 else (n_act + L - 1) // L
                lax.fori_loop(0, nvec, acc, 0)

                def scan(i, c, target=target):
                    cnt_above, bsel, m = c
                    bn = 15 - i
                    cv = jnp.sum(hist[pl.ds(bn * L, L)])
                    hit = (cnt_above < target) & (cnt_above + cv >= target)
                    bsel = jnp.where(hit, bn, bsel)
                    m = jnp.where(hit, target - cnt_above, m)
                    return (cnt_above + cv, bsel, m)
                _, bsel, m = lax.fori_loop(0, 16, scan,
                                           (jnp.int32(0), jnp.int32(0),
                                            jnp.int32(0)))
                bu = lax.convert_element_type(bsel, jnp.uint32)

                if p < 7:
                    def comp(i, cnt, n_act=n_act, src=src, sh=sh, bu=bu, p=p):
                        kv = src[pl.ds(i * L, L)]
                        if p > 0:
                            kv = plsc.bitcast(kv, jnp.uint32)
                        in_rng = (i * L + lane) < n_act
                        dig = lax.shift_right_logical(kv, jnp.uint32(sh)) \
                            & jnp.uint32(0xF)
                        sel = in_rng & (dig == bu)
                        si = jnp.where(sel, 1, 0)
                        dst = cnt + plsc.cumsum(si) - si
                        plsc.store_scatter(act, [dst],
                                           plsc.bitcast(kv, jnp.int32),
                                           mask=sel)
                        return cnt + jnp.sum(si)
                    n_act = lax.fori_loop(0, nvec, comp, jnp.int32(0))
                tkey = tkey | lax.shift_left(bu, jnp.uint32(sh))
                target = m
            m = target

            def sel(i, c):
                n_sel, n_eq = c
                kv = keys[pl.ds(i * L, L)]
                gt = kv > tkey
                eq = kv == tkey
                eqi = jnp.where(eq, 1, 0)
                pre = plsc.cumsum(eqi) - eqi
                take = gt | (eq & ((n_eq + pre) < m))
                ti = jnp.where(take, 1, 0)
                dst = n_sel + plsc.cumsum(ti) - ti
                gidx16 = b * T + i * L + lane
                plsc.store_scatter(selidx, [dst], gidx16, mask=take)
                return (n_sel + jnp.sum(ti), n_eq + jnp.sum(eqi))
            lax.fori_loop(0, NV, sel, (0, 0))
            pltpu.sync_copy(selidx, shared_idx.at[b])

        plsc.subcore_barrier()

        g = sid * NC + cid
        b2 = g // 8
        part = (g % 8) * GROWS
        pltpu.sync_copy(shared_idx.at[b2, pl.ds(part, GROWS)], gidx)
        pltpu.async_copy(hid_hbm.at[gidx], mrows, sem).wait()
        pltpu.sync_copy(mrows, mem_hbm.at[b2, pl.ds(part, GROWS)])

    return k(scores, hidden)


def _attn_logits(memory, query, Wq, bqr, Wo, bor):
    NVB = 3200
    grid = (VOCAB // NVB,)

    def body(mem_ref, q_ref, wq_ref, bq_ref, wo_ref, bo_ref, out_ref, ctx_ref):
        @pl.when(pl.program_id(0) == 0)
        def _():
            q = lax.dot_general(
                q_ref[...], wq_ref[...], (((1,), (0,)), ((), ())),
                preferred_element_type=jnp.float32,
                precision=_PREC) + bq_ref[...]
            ctxs = []
            for b in range(BATCH):
                mb = mem_ref[b]
                qb = q[b:b + 1, :]
                s = jnp.sum(_r16(mb) * _r16(qb), axis=-1,
                            keepdims=True)
                s = s - jnp.max(s, axis=0, keepdims=True)
                e = jnp.exp(s)
                attn = e / jnp.sum(e, axis=0, keepdims=True)
                ctxs.append(jnp.sum(attn * mb, axis=0, keepdims=True))
            ctx_ref[...] = jnp.concatenate(ctxs, axis=0)
        out_ref[...] = lax.dot_general(
            ctx_ref[...], wo_ref[...], (((1,), (0,)), ((), ())),
            preferred_element_type=jnp.float32,
            precision=_PREC) + bo_ref[...]

    return pl.pallas_call(
        body,
        grid=grid,
        in_specs=[
            pl.BlockSpec((BATCH, K, HID), lambda i: (0, 0, 0)),
            pl.BlockSpec((BATCH, HID), lambda i: (0, 0)),
            pl.BlockSpec((HID, HID), lambda i: (0, 0)),
            pl.BlockSpec((1, HID), lambda i: (0, 0)),
            pl.BlockSpec((HID, NVB), lambda i: (0, i)),
            pl.BlockSpec((1, NVB), lambda i: (0, i)),
        ],
        out_specs=pl.BlockSpec((BATCH, NVB), lambda i: (0, i)),
        out_shape=jax.ShapeDtypeStruct((BATCH, VOCAB), jnp.float32),
        scratch_shapes=[pltpu.VMEM((BATCH, HID), jnp.float32)],
    )(memory, query, Wq, bqr, Wo, bor)


def kernel(seq, embed, W1, b1, W2, b2, gamma, beta, Wg, bg, Wq, bq, Wo, bo):
    seq3 = seq.astype(jnp.int32).reshape(NW, 8, NTOK // NW // 8)
    h = _embed_gather(seq3, embed)
    hidden, scores = _encoder(
        h, W1, b1.reshape(1, -1), W2, b2.reshape(1, -1),
        gamma.reshape(1, -1), beta.reshape(1, -1), Wg.reshape(1, -1))
    memory = _topk_gather(scores.reshape(BATCH, T), hidden)
    query = hidden.reshape(BATCH, T, HID)[:, -2, :]
    return _attn_logits(memory, query, Wq, bq.reshape(1, -1),
                        Wo, bo.reshape(1, -1))


# speedup vs baseline: 1.2751x; 1.0010x over previous
import functools

import jax
import jax.numpy as jnp
from jax import lax
from jax.experimental import pallas as pl
from jax.experimental.pallas import tpu as pltpu
from jax.experimental.pallas import tpu_sc as plsc

VOCAB = 32000
HID = 1024
T = 2048
BATCH = 4
NTOK = BATCH * T
K = 256
NCAND = T - 3
NC, NS, L = 2, 16, 16
NW = NC * NS

_PREC = lax.Precision.DEFAULT


def _r16(x):
    return x.astype(jnp.bfloat16).astype(jnp.float32)


def _embed_gather(seq3, embed):
    n_ch, ch = seq3.shape[1], seq3.shape[2]
    tok_per_w = n_ch * ch
    mesh = plsc.VectorSubcoreMesh(core_axis_name="c", subcore_axis_name="s")

    @functools.partial(
        pl.kernel,
        out_type=jax.ShapeDtypeStruct((NTOK, HID), jnp.float32),
        mesh=mesh,
        scratch_types=[
            pltpu.VMEM((n_ch, ch), jnp.int32),
            pltpu.VMEM((ch, HID), jnp.float32),
            pltpu.VMEM((ch, HID), jnp.float32),
            pltpu.VMEM((ch, HID), jnp.float32),
            pltpu.SemaphoreType.DMA,
            pltpu.SemaphoreType.DMA,
            pltpu.SemaphoreType.DMA,
            pltpu.SemaphoreType.DMA,
            pltpu.SemaphoreType.DMA,
            pltpu.SemaphoreType.DMA,
        ],
        compiler_params=pltpu.CompilerParams(needs_layout_passes=False),
    )
    def k(seq_hbm, embed_hbm, h_hbm, idx_v, rows0, rows1, rows2,
          g0, g1, g2, s0, s1, s2):
        wid = lax.axis_index("s") * NC + lax.axis_index("c")
        base = wid * tok_per_w
        pltpu.sync_copy(seq_hbm.at[wid], idx_v)
        rows = (rows0, rows1, rows2)
        gsem = (g0, g1, g2)
        ssem = (s0, s1, s2)
        cps = [None] * n_ch
        outs = [None] * n_ch
        cps[0] = pltpu.async_copy(embed_hbm.at[idx_v.at[0]], rows[0], gsem[0])
        cps[1] = pltpu.async_copy(embed_hbm.at[idx_v.at[1]], rows[1], gsem[1])
        for j in range(n_ch):
            p = j % 3
            if j + 2 < n_ch:
                if outs[j - 1] is not None:
                    outs[j - 1].wait()
                cps[j + 2] = pltpu.async_copy(
                    embed_hbm.at[idx_v.at[j + 2]], rows[(j + 2) % 3],
                    gsem[(j + 2) % 3])
            cps[j].wait()
            outs[j] = pltpu.async_copy(
                rows[p], h_hbm.at[pl.ds(base + j * ch, ch)], ssem[p])
        outs[n_ch - 3].wait()
        outs[n_ch - 2].wait()
        outs[n_ch - 1].wait()

    return k(seq3, embed)


def _encoder(h, W1, b1r, W2, b2r, gammar, betar, wgr):
    BT = 1024
    grid = (NTOK // BT,)

    def body(h_ref, w1_ref, b1_ref, w2_ref, b2_ref, g_ref, be_ref, wg_ref,
             hid_ref, sc_ref):
        hb = h_ref[...]
        a1 = lax.dot_general(hb, w1_ref[...], (((1,), (0,)), ((), ())),
                             preferred_element_type=jnp.float32,
                             precision=_PREC)
        a1 = jnp.maximum(a1 + b1_ref[...], 0.0)
        x = lax.dot_general(a1, w2_ref[...], (((1,), (0,)), ((), ())),
                            preferred_element_type=jnp.float32,
                            precision=_PREC)
        x = x + b2_ref[...] + hb
        mu = jnp.mean(x, axis=-1, keepdims=True)
        xc = x - mu
        var = jnp.mean(xc * xc, axis=-1, keepdims=True)
        ln = xc * lax.rsqrt(var + 1e-5) * g_ref[...] + be_ref[...]
        hid_ref[...] = ln
        sc_ref[...] = jnp.sum(_r16(ln) * _r16(wg_ref[...]), axis=-1,
                              keepdims=True)

    hidden, scores = pl.pallas_call(
        body,
        grid=grid,
        in_specs=[
            pl.BlockSpec((BT, HID), lambda i: (i, 0)),
            pl.BlockSpec((HID, 2 * HID), lambda i: (0, 0)),
            pl.BlockSpec((1, 2 * HID), lambda i: (0, 0)),
            pl.BlockSpec((2 * HID, HID), lambda i: (0, 0)),
            pl.BlockSpec((1, HID), lambda i: (0, 0)),
            pl.BlockSpec((1, HID), lambda i: (0, 0)),
            pl.BlockSpec((1, HID), lambda i: (0, 0)),
            pl.BlockSpec((1, HID), lambda i: (0, 0)),
        ],
        out_specs=[
            pl.BlockSpec((BT, HID), lambda i: (i, 0)),
            pl.BlockSpec((BT, 1), lambda i: (i, 0)),
        ],
        out_shape=[
            jax.ShapeDtypeStruct((NTOK, HID), jnp.float32),
            jax.ShapeDtypeStruct((NTOK, 1), jnp.float32),
        ],
    )(h, W1, b1r, W2, b2r, gammar, betar, wgr)
    return hidden, scores


def _topk_gather(scores, hidden):
    mesh = plsc.VectorSubcoreMesh(core_axis_name="c", subcore_axis_name="s")
    NV = T // L
    GROWS = K // 8

    @functools.partial(
        pl.kernel,
        out_type=jax.ShapeDtypeStruct((BATCH, K, HID), jnp.float32),
        mesh=mesh,
        scratch_types=[
            pltpu.VMEM((T,), jnp.float32),
            pltpu.VMEM((T,), jnp.uint32),
            pltpu.VMEM((T,), jnp.int32),
            pltpu.VMEM((16 * L,), jnp.int32),
            pltpu.VMEM((K,), jnp.int32),
            pltpu.VMEM((GROWS,), jnp.int32),
            pltpu.VMEM((GROWS, HID), jnp.float32),
            pltpu.VMEM_SHARED((BATCH, K), jnp.int32),
            pltpu.SemaphoreType.DMA,
        ],
        compiler_params=pltpu.CompilerParams(needs_layout_passes=False),
    )
    def k(sc_hbm, hid_hbm, mem_hbm, srow, keys, act, hist, selidx, gidx,
          mrows, shared_idx, sem):
        cid = lax.axis_index("c")
        sid = lax.axis_index("s")
        lane = lax.iota(jnp.int32, L)
        zero16 = jnp.zeros((L,), jnp.int32)
        ones16 = jnp.ones((L,), jnp.int32)

        @pl.when(sid < BATCH)
        def _phase_a():
            b = sid
            pltpu.sync_copy(sc_hbm.at[b], srow)

            def mkkeys(i, _):
                sv = srow[pl.ds(i * L, L)]
                ki = plsc.bitcast(sv, jnp.int32)
                sgn = lax.shift_right_arithmetic(ki, 31)
                key = plsc.bitcast(ki, jnp.uint32) ^ (
                    plsc.bitcast(sgn, jnp.uint32) | jnp.uint32(0x80000000))
                valid = (i * L + lane) < NCAND
                keys[pl.ds(i * L, L)] = jnp.where(valid, key, jnp.uint32(0))
                return 0
            lax.fori_loop(0, NV, mkkeys, 0)

            tkey = jnp.uint32(0)
            target = jnp.int32(K)
            n_act = jnp.int32(T)
            for p in range(8):
                sh = 28 - 4 * p
                src = keys if p == 0 else act

                def zh(i, _):
                    hist[pl.ds(i * L, L)] = zero16
                    return 0
                lax.fori_loop(0, 16, zh, 0)

                def acc(i, _, n_act=n_act, src=src, sh=sh, p=p):
                    kv = src[pl.ds(i * L, L)]
                    if p > 0:
                        kv = plsc.bitcast(kv, jnp.uint32)
                    in_rng = (i * L + lane) < n_act
                    dig = lax.convert_element_type(
                        lax.shift_right_logical(kv, jnp.uint32(sh))
                        & jnp.uint32(0xF), jnp.int32)
                    plsc.addupdate_scatter(hist, [dig * L + lane], ones16,
                                           mask=in_rng)
                    return 0
                nvec = NV if p == 0 else (n_act + L - 1) // L
                lax.fori_loop(0, nvec, acc, 0)

                def scan(i, c, target=target):
                    cnt_above, bsel, m = c
                    bn = 15 - i
                    cv = jnp.sum(hist[pl.ds(bn * L, L)])
                    hit = (cnt_above < target) & (cnt_above + cv >= target)
                    bsel = jnp.where(hit, bn, bsel)
                    m = jnp.where(hit, target - cnt_above, m)
                    return (cnt_above + cv, bsel, m)
                _, bsel, m = lax.fori_loop(0, 16, scan,
                                           (jnp.int32(0), jnp.int32(0),
                                            jnp.int32(0)))
                bu = lax.convert_element_type(bsel, jnp.uint32)

                if p < 7:
                    def comp(i, cnt, n_act=n_act, src=src, sh=sh, bu=bu, p=p):
                        kv = src[pl.ds(i * L, L)]
                        if p > 0:
                            kv = plsc.bitcast(kv, jnp.uint32)
                        in_rng = (i * L + lane) < n_act
                        dig = lax.shift_right_logical(kv, jnp.uint32(sh)) \
                            & jnp.uint32(0xF)
                        sel = in_rng & (dig == bu)
                        si = jnp.where(sel, 1, 0)
                        dst = cnt + plsc.cumsum(si) - si
                        plsc.store_scatter(act, [dst],
                                           plsc.bitcast(kv, jnp.int32),
                                           mask=sel)
                        return cnt + jnp.sum(si)
                    n_act = lax.fori_loop(0, nvec, comp, jnp.int32(0))
                tkey = tkey | lax.shift_left(bu, jnp.uint32(sh))
                target = m
            m = target

            def sel(i, c):
                n_sel, n_eq = c
                kv = keys[pl.ds(i * L, L)]
                gt = kv > tkey
                eq = kv == tkey
                eqi = jnp.where(eq, 1, 0)
                pre = plsc.cumsum(eqi) - eqi
                take = gt | (eq & ((n_eq + pre) < m))
                ti = jnp.where(take, 1, 0)
                dst = n_sel + plsc.cumsum(ti) - ti
                gidx16 = b * T + i * L + lane
                plsc.store_scatter(selidx, [dst], gidx16, mask=take)
                return (n_sel + jnp.sum(ti), n_eq + jnp.sum(eqi))
            lax.fori_loop(0, NV, sel, (0, 0))
            pltpu.sync_copy(selidx, shared_idx.at[b])

        plsc.subcore_barrier()

        g = sid * NC + cid
        b2 = g // 8
        part = (g % 8) * GROWS
        pltpu.sync_copy(shared_idx.at[b2, pl.ds(part, GROWS)], gidx)
        pltpu.async_copy(hid_hbm.at[gidx], mrows, sem).wait()
        pltpu.sync_copy(mrows, mem_hbm.at[b2, pl.ds(part, GROWS)])

    return k(scores, hidden)


def _attn_logits(memory, query, Wq, bqr, Wo, bor):
    NVB = 3200
    grid = (VOCAB // NVB,)

    def body(mem_ref, q_ref, wq_ref, bq_ref, wo_ref, bo_ref, out_ref, ctx_ref):
        @pl.when(pl.program_id(0) == 0)
        def _():
            q = lax.dot_general(
                q_ref[...], wq_ref[...], (((1,), (0,)), ((), ())),
                preferred_element_type=jnp.float32,
                precision=_PREC) + bq_ref[...]
            ctxs = []
            for b in range(BATCH):
                mb = mem_ref[b]
                qb = q[b:b + 1, :]
                s = jnp.sum(_r16(mb) * _r16(qb), axis=-1,
                            keepdims=True)
                s = s - jnp.max(s, axis=0, keepdims=True)
                e = jnp.exp(s)
                attn = e / jnp.sum(e, axis=0, keepdims=True)
                ctxs.append(jnp.sum(attn * mb, axis=0, keepdims=True))
            ctx_ref[...] = jnp.concatenate(ctxs, axis=0)
        out_ref[...] = lax.dot_general(
            ctx_ref[...], wo_ref[...], (((1,), (0,)), ((), ())),
            preferred_element_type=jnp.float32,
            precision=_PREC) + bo_ref[...]

    return pl.pallas_call(
        body,
        grid=grid,
        in_specs=[
            pl.BlockSpec((BATCH, K, HID), lambda i: (0, 0, 0)),
            pl.BlockSpec((BATCH, HID), lambda i: (0, 0)),
            pl.BlockSpec((HID, HID), lambda i: (0, 0)),
            pl.BlockSpec((1, HID), lambda i: (0, 0)),
            pl.BlockSpec((HID, NVB), lambda i: (0, i)),
            pl.BlockSpec((1, NVB), lambda i: (0, i)),
        ],
        out_specs=pl.BlockSpec((BATCH, NVB), lambda i: (0, i)),
        out_shape=jax.ShapeDtypeStruct((BATCH, VOCAB), jnp.float32),
        scratch_shapes=[pltpu.VMEM((BATCH, HID), jnp.float32)],
    )(memory, query, Wq, bqr, Wo, bor)


def kernel(seq, embed, W1, b1, W2, b2, gamma, beta, Wg, bg, Wq, bq, Wo, bo):
    seq3 = seq.astype(jnp.int32).reshape(NW, 8, NTOK // NW // 8)
    h = _embed_gather(seq3, embed)
    hidden, scores = _encoder(
        h, W1, b1.reshape(1, -1), W2, b2.reshape(1, -1),
        gamma.reshape(1, -1), beta.reshape(1, -1), Wg.reshape(1, -1))
    memory = _topk_gather(scores.reshape(BATCH, T), hidden)
    query = hidden.reshape(BATCH, T, HID)[:, -2, :]
    return _attn_logits(memory, query, Wq, bq.reshape(1, -1),
                        Wo, bo.reshape(1, -1))
